# baseline (speedup 1.0000x reference)
import functools

import jax
import jax.numpy as jnp
from jax import lax
from jax.experimental import pallas as pl
from jax.experimental.pallas import tpu as pltpu
from jax.experimental.pallas import tpu_sc as plsc

NC = 2
NS = 16
NW = NC * NS
L = 16

D = 64
G = 400


def _sc_avg_gather(idxf_hbm, idxt_hbm, from_hbm, to_hbm, out_hbm,
                   idxf_v, idxt_v, rows_f, rows_t,
                   gsf0, gsf1, gst0, gst1, ws0, ws1):
    ngath = idxf_hbm.shape[1]
    wid = lax.axis_index("s") * NC + lax.axis_index("c")
    base_g = wid * ngath
    gsf = [gsf0, gsf1]
    gst = [gst0, gst1]
    ws = [ws0, ws1]

    pltpu.sync_copy(idxf_hbm.at[wid], idxf_v)
    pltpu.sync_copy(idxt_hbm.at[wid], idxt_v)

    def issue_gathers(j, p):
        pltpu.async_copy(from_hbm.at[idxf_v.at[j]], rows_f.at[p], gsf[p])
        pltpu.async_copy(to_hbm.at[idxt_v.at[j]], rows_t.at[p], gst[p])

    def wait_gathers(j, p):
        pltpu.make_async_copy(from_hbm.at[idxf_v.at[j]], rows_f.at[p], gsf[p]).wait()
        pltpu.make_async_copy(to_hbm.at[idxt_v.at[j]], rows_t.at[p], gst[p]).wait()

    def out_slice(j):
        return out_hbm.at[pl.ds((base_g + j) * G, G)]

    def compute(p):
        rf = rows_f.at[p]
        rt = rows_t.at[p]

        def row(r, c2):
            for c in range(D // L):
                a = rf[r, pl.ds(c * L, L)]
                b = rt[r, pl.ds(c * L, L)]
                rf[r, pl.ds(c * L, L)] = a + b
            return c2

        lax.fori_loop(0, G, row, 0)

    issue_gathers(0, 0)

    def step(j2, carry):
        j = 2 * j2
        wait_gathers(j, 0)

        @pl.when(j2 > 0)
        def _():
            pltpu.make_async_copy(rows_f.at[1], out_slice(j - 1), ws[1]).wait()

        issue_gathers(j + 1, 1)
        compute(0)
        pltpu.async_copy(rows_f.at[0], out_slice(j), ws[0])

        wait_gathers(j + 1, 1)
        pltpu.make_async_copy(rows_f.at[0], out_slice(j), ws[0]).wait()

        @pl.when(j2 < ngath // 2 - 1)
        def _():
            issue_gathers(j + 2, 0)

        compute(1)
        pltpu.async_copy(rows_f.at[1], out_slice(j + 1), ws[1])
        return carry

    lax.fori_loop(0, ngath // 2, step, 0)
    pltpu.make_async_copy(rows_f.at[1], out_slice(ngath - 1), ws[1]).wait()


@jax.jit
def _run(idx_f, idx_t, topo_from, topo_to):
    ngath = idx_f.shape[1]
    B = NW * ngath * G
    mesh = plsc.VectorSubcoreMesh(core_axis_name="c", subcore_axis_name="s",
                                  num_cores=NC, num_subcores=NS)
    k = pl.kernel(
        _sc_avg_gather,
        out_type=jax.ShapeDtypeStruct((B, D), jnp.float32),
        mesh=mesh,
        compiler_params=pltpu.CompilerParams(use_tc_tiling_on_sc=False),
        scratch_types=[
            pltpu.VMEM((ngath, G), jnp.int32),
            pltpu.VMEM((ngath, G), jnp.int32),
            pltpu.VMEM((2, G, D), jnp.float32),
            pltpu.VMEM((2, G, D), jnp.float32),
            pltpu.SemaphoreType.DMA,
            pltpu.SemaphoreType.DMA,
            pltpu.SemaphoreType.DMA,
            pltpu.SemaphoreType.DMA,
            pltpu.SemaphoreType.DMA,
            pltpu.SemaphoreType.DMA,
        ],
    )
    return k(idx_f, idx_t, topo_from, topo_to)


def kernel(input, positions, topo_from, topo_to):
    Bt, T, _ = positions.shape
    B = Bt * T
    n = topo_from.shape[0]
    idx_f = (positions[:, :, -2] % n).reshape(NW, B // (NW * G), G).astype(jnp.int32)
    idx_t = (positions[:, :, -1] % n).reshape(NW, B // (NW * G), G).astype(jnp.int32)
    out = _run(idx_f, idx_t, topo_from * 0.5, topo_to * 0.5)
    return out.reshape(Bt, T, D)


# speedup vs baseline: 1.2138x; 1.2138x over previous
import functools

import jax
import jax.numpy as jnp
from jax import lax
from jax.experimental import pallas as pl
from jax.experimental.pallas import tpu as pltpu
from jax.experimental.pallas import tpu_sc as plsc

NC = 2
NS = 16
NW = NC * NS
L = 16

D = 64
T = 50
BPC = 8
G = BPC * T


def _sc_avg_gather(idxf_hbm, idxt_hbm, from_hbm, to_hbm, out_hbm,
                   idxf_v, idxt_v, rows_f, rows_t,
                   gsf0, gsf1, gst0, gst1, ws0, ws1):
    ngath = idxf_hbm.shape[1]
    wid = lax.axis_index("s") * NC + lax.axis_index("c")
    base_g = wid * ngath
    gsf = [gsf0, gsf1]
    gst = [gst0, gst1]
    ws = [ws0, ws1]

    pltpu.sync_copy(idxf_hbm.at[wid], idxf_v)
    pltpu.sync_copy(idxt_hbm.at[wid], idxt_v)

    def issue_gathers(j, p):
        pltpu.async_copy(from_hbm.at[idxf_v.at[j]], rows_f.at[p], gsf[p])
        pltpu.async_copy(to_hbm.at[idxt_v.at[j]], rows_t.at[p], gst[p])

    def wait_gathers(j, p):
        pltpu.make_async_copy(from_hbm.at[idxf_v.at[j]], rows_f.at[p], gsf[p]).wait()
        pltpu.make_async_copy(to_hbm.at[idxt_v.at[j]], rows_t.at[p], gst[p]).wait()

    def issue_writeout(j, p):
        b0 = (base_g + j) * BPC
        for i in range(BPC):
            pltpu.async_copy(rows_f.at[p, pl.ds(i * T, T)], out_hbm.at[b0 + i],
                             ws[p])

    def wait_writeout(j, p):
        b0 = (base_g + j) * BPC
        for i in range(BPC):
            pltpu.make_async_copy(rows_f.at[p, pl.ds(i * T, T)],
                                  out_hbm.at[b0 + i], ws[p]).wait()

    def compute(p):
        rf = rows_f.at[p]
        rt = rows_t.at[p]

        def row(r, c2):
            for c in range(D // L):
                a = rf[r, pl.ds(c * L, L)]
                b = rt[r, pl.ds(c * L, L)]
                rf[r, pl.ds(c * L, L)] = (a + b) * 0.5
            return c2

        lax.fori_loop(0, G, row, 0)

    issue_gathers(0, 0)

    def step(j2, carry):
        j = 2 * j2
        wait_gathers(j, 0)

        @pl.when(j2 > 0)
        def _():
            wait_writeout(j - 1, 1)

        issue_gathers(j + 1, 1)
        compute(0)
        issue_writeout(j, 0)

        wait_gathers(j + 1, 1)
        wait_writeout(j, 0)

        @pl.when(j2 < ngath // 2 - 1)
        def _():
            issue_gathers(j + 2, 0)

        compute(1)
        issue_writeout(j + 1, 1)
        return carry

    lax.fori_loop(0, ngath // 2, step, 0)
    wait_writeout(ngath - 1, 1)


@jax.jit
def _run(idx_f, idx_t, topo_from, topo_to):
    ngath = idx_f.shape[1]
    Bt = NW * ngath * BPC
    mesh = plsc.VectorSubcoreMesh(core_axis_name="c", subcore_axis_name="s",
                                  num_cores=NC, num_subcores=NS)
    k = pl.kernel(
        _sc_avg_gather,
        out_type=jax.ShapeDtypeStruct((Bt, T, D), jnp.float32),
        mesh=mesh,
        compiler_params=pltpu.CompilerParams(use_tc_tiling_on_sc=False),
        scratch_types=[
            pltpu.VMEM((ngath, G), jnp.int32),
            pltpu.VMEM((ngath, G), jnp.int32),
            pltpu.VMEM((2, G, D), jnp.float32),
            pltpu.VMEM((2, G, D), jnp.float32),
            pltpu.SemaphoreType.DMA,
            pltpu.SemaphoreType.DMA,
            pltpu.SemaphoreType.DMA,
            pltpu.SemaphoreType.DMA,
            pltpu.SemaphoreType.DMA,
            pltpu.SemaphoreType.DMA,
        ],
    )
    return k(idx_f, idx_t, topo_from, topo_to)


def kernel(input, positions, topo_from, topo_to):
    Bt, Tt, _ = positions.shape
    B = Bt * Tt
    idx_f = positions[:, :, -2].reshape(NW, B // (NW * G), G).astype(jnp.int32)
    idx_t = positions[:, :, -1].reshape(NW, B // (NW * G), G).astype(jnp.int32)
    return _run(idx_f, idx_t, topo_from, topo_to)
